# Initial kernel scaffold; baseline (speedup 1.0000x reference)
import jax
import jax.numpy as jnp
from jax.experimental import pallas as pl


def kernel(x, table):
    raise NotImplementedError("write your pallas kernel here")


# speedup vs baseline: 4.7567x; 4.7567x over previous
import functools

import jax
import jax.numpy as jnp
from jax import lax
from jax.experimental import pallas as pl
from jax.experimental.pallas import tpu as pltpu
from jax.experimental.pallas import tpu_sc as plsc

_VOCAB = 1000000
_D = 32
_CHUNK = 1024
_LANES = 16


def kernel(x, table):
    B0, S = x.shape
    B = B0 * S
    V, D = table.shape
    xf = x.reshape(B)

    info = plsc.get_sparse_core_info()
    NC, NS = info.num_cores, info.num_subcores
    NW = NC * NS
    b_per_w = B // NW
    n_chunks = b_per_w // _CHUNK
    assert b_per_w * NW == B and n_chunks * _CHUNK == b_per_w

    mesh = plsc.VectorSubcoreMesh(core_axis_name="c", subcore_axis_name="s")

    @functools.partial(
        pl.kernel,
        mesh=mesh,
        out_type=jax.ShapeDtypeStruct((B, D), jnp.float32),
        scratch_types=[
            pltpu.VMEM((_CHUNK,), jnp.int32),
            pltpu.VMEM((_CHUNK, _D), jnp.float32),
            pltpu.SemaphoreType.DMA,
        ],
        compiler_params=pltpu.CompilerParams(use_tc_tiling_on_sc=False),
    )
    def emb(idx_hbm, table_hbm, out_hbm, idx_v, rows_v, sem):
        wid = lax.axis_index("s") * NC + lax.axis_index("c")
        base = wid * b_per_w

        def body(g, carry):
            off = base + g * _CHUNK
            pltpu.sync_copy(idx_hbm.at[pl.ds(off, _CHUNK)], idx_v)

            def clamp(i, c):
                v = idx_v[pl.ds(i * _LANES, _LANES)]
                ok = (v >= 0) & (v < V)
                idx_v[pl.ds(i * _LANES, _LANES)] = jnp.where(ok, v, 0)
                return c

            lax.fori_loop(0, _CHUNK // _LANES, clamp, 0)
            pltpu.async_copy(table_hbm.at[idx_v], rows_v, sem).wait()
            pltpu.sync_copy(rows_v, out_hbm.at[pl.ds(off, _CHUNK)])
            return carry

        lax.fori_loop(0, n_chunks, body, 0)

    out = emb(xf, table)
    return out.reshape(B0, S, D)


# speedup vs baseline: 5.0430x; 1.0602x over previous
import functools

import jax
import jax.numpy as jnp
from jax import lax
from jax.experimental import pallas as pl
from jax.experimental.pallas import tpu as pltpu
from jax.experimental.pallas import tpu_sc as plsc

_D = 32
_CHUNK = 1024
_LANES = 16
_NBUF = 2


def kernel(x, table):
    B0, S = x.shape
    B = B0 * S
    V, D = table.shape
    xf = x.reshape(B)

    info = plsc.get_sparse_core_info()
    NC, NS = info.num_cores, info.num_subcores
    NW = NC * NS
    b_per_w = B // NW
    n_chunks = b_per_w // _CHUNK
    assert b_per_w * NW == B and n_chunks * _CHUNK == b_per_w
    assert n_chunks % 2 == 0 and n_chunks >= 4
    n_pairs = n_chunks // 2

    mesh = plsc.VectorSubcoreMesh(core_axis_name="c", subcore_axis_name="s")

    @functools.partial(
        pl.kernel,
        mesh=mesh,
        out_type=jax.ShapeDtypeStruct((B, D), jnp.float32),
        scratch_types=[
            pltpu.VMEM((_CHUNK,), jnp.int32),
            pltpu.VMEM((_CHUNK,), jnp.int32),
            pltpu.VMEM((_CHUNK, _D), jnp.float32),
            pltpu.VMEM((_CHUNK, _D), jnp.float32),
            pltpu.SemaphoreType.DMA,
            pltpu.SemaphoreType.DMA,
            pltpu.SemaphoreType.DMA,
            pltpu.SemaphoreType.DMA,
            pltpu.SemaphoreType.DMA,
            pltpu.SemaphoreType.DMA,
        ],
        compiler_params=pltpu.CompilerParams(use_tc_tiling_on_sc=False),
    )
    def emb(idx_hbm, table_hbm, out_hbm, idx_v0, idx_v1, rows_v0, rows_v1,
            isem0, isem1, gsem0, gsem1, ssem0, ssem1):
        idxs = (idx_v0, idx_v1)
        rows = (rows_v0, rows_v1)
        isems = (isem0, isem1)
        gsems = (gsem0, gsem1)
        ssems = (ssem0, ssem1)
        wid = lax.axis_index("s") * NC + lax.axis_index("c")
        base = wid * b_per_w

        def idx_start(g, b):
            pltpu.async_copy(
                idx_hbm.at[pl.ds(base + g * _CHUNK, _CHUNK)], idxs[b],
                isems[b])

        def idx_wait(b):
            pltpu.make_async_copy(
                idx_hbm.at[pl.ds(base, _CHUNK)], idxs[b],
                isems[b]).wait()

        def clamp(b):
            def one(i, c):
                v = idxs[b][pl.ds(i * _LANES, _LANES)]
                ok = (v >= 0) & (v < V)
                idxs[b][pl.ds(i * _LANES, _LANES)] = jnp.where(ok, v, 0)
                return c

            lax.fori_loop(0, _CHUNK // _LANES, one, 0)

        def gather_start(b):
            pltpu.async_copy(table_hbm.at[idxs[b]], rows[b], gsems[b])

        def gather_wait(b):
            pltpu.make_async_copy(table_hbm.at[idxs[b]], rows[b],
                                  gsems[b]).wait()

        def store_start(g, b):
            pltpu.async_copy(
                rows[b], out_hbm.at[pl.ds(base + g * _CHUNK, _CHUNK)],
                ssems[b])

        def store_wait(b):
            pltpu.make_async_copy(
                rows[b], out_hbm.at[pl.ds(base, _CHUNK)],
                ssems[b]).wait()

        idx_start(0, 0)
        idx_start(1, 1)
        idx_wait(0)
        clamp(0)
        gather_start(0)
        idx_wait(1)
        clamp(1)
        gather_start(1)
        gather_wait(0)
        store_start(0, 0)
        idx_start(2, 0)

        def pair(gg, carry):
            def chunk(g, b, last):
                idx_wait(b)
                clamp(b)
                store_wait(b)
                gather_start(b)
                gather_wait(1 - b)
                store_start(g - 1, 1 - b)

                @pl.when(jnp.logical_not(last))
                def _():
                    idx_start(g + 1, 1 - b)

            g0 = gg * 2
            chunk(g0, 0, jnp.bool_(False))
            chunk(g0 + 1, 1, gg >= n_pairs - 1)
            return carry

        lax.fori_loop(1, n_pairs, pair, 0)

        gather_wait(1)
        store_start(n_chunks - 1, 1)
        store_wait(0)
        store_wait(1)

    out = emb(xf, table)
    return out.reshape(B0, S, D)
